# baseline (speedup 1.0000x reference)
import functools

import jax
import jax.numpy as jnp
from jax import lax
from jax.experimental import pallas as pl
from jax.experimental.pallas import tpu as pltpu
from jax.experimental.pallas import tpu_sc as plsc

N_NODES = 10000
D = 128
N_EDGES = 320000

NC = 2
NS = 16
NW = NC * NS

CHUNK = 128
CPT = 80
EPT = CHUNK * CPT
E_PAD = EPT * NW
ACC_ROWS = 10240
ZROWS = ACC_ROWS // NS
ROWS_OUT = N_NODES // NS
N_PAD = 10016
TOTAL_CHUNKS = N_EDGES // CHUNK
CPT0 = 160
CPT1 = 0
NBUF = 2
IBLK = 16
NIB = CPT // IBLK

_mesh = plsc.VectorSubcoreMesh(
    core_axis_name="c", subcore_axis_name="s", num_cores=NC, num_subcores=NS)


@functools.partial(
    pl.kernel,
    out_type=jax.ShapeDtypeStruct((NC, NS, ROWS_OUT, D), jnp.float32),
    mesh=_mesh,
    scratch_types=[
        pltpu.VMEM((CPT, CHUNK), jnp.int32),
        pltpu.VMEM((CHUNK, D), jnp.float32),
        pltpu.VMEM_SHARED((ACC_ROWS, D), jnp.float32),
    ],
)
def _sc_degree(dst_hbm, ones_hbm, zeros_hbm, out_hbm, idx_v, ones_v, acc):
    c = lax.axis_index("c")
    s = lax.axis_index("s")
    w = c * NS + s
    nreal = jnp.minimum(CPT, jnp.maximum(0, TOTAL_CHUNKS - w * CPT))
    pltpu.sync_copy(dst_hbm.at[pl.ds(w * CPT, CPT)], idx_v)
    pltpu.sync_copy(ones_hbm, ones_v)
    pltpu.sync_copy(zeros_hbm, acc.at[pl.ds(s * ZROWS, ZROWS)])
    plsc.subcore_barrier()

    def body(j, _):
        pltpu.sync_copy(ones_v, acc.at[idx_v.at[j]], add=True)
        return _

    lax.fori_loop(0, nreal, body, None)
    plsc.subcore_barrier()
    pltpu.sync_copy(acc.at[pl.ds(s * ROWS_OUT, ROWS_OUT)], out_hbm.at[c, s])


@functools.partial(
    pl.kernel,
    out_type=jax.ShapeDtypeStruct((NC, NS, ROWS_OUT, D), jnp.float32),
    mesh=_mesh,
    scratch_types=[
        pltpu.VMEM((IBLK, CHUNK), jnp.int32),
        pltpu.VMEM((IBLK, CHUNK), jnp.int32),
        pltpu.VMEM((CHUNK, D), jnp.float32),
        pltpu.VMEM((CHUNK, D), jnp.float32),
        pltpu.SemaphoreType.DMA,
        pltpu.SemaphoreType.DMA,
        pltpu.VMEM_SHARED((ACC_ROWS, D), jnp.float32),
    ],
)
def _sc_aggregate(g_hbm, src_hbm, dst_hbm, zeros_hbm, out_hbm,
                  sidx, didx, r0, r1, sm0, sm1, acc):
    rows = [r0, r1]
    sems = [sm0, sm1]
    c = lax.axis_index("c")
    s = lax.axis_index("s")
    pltpu.sync_copy(zeros_hbm, acc.at[pl.ds(s * ZROWS, ZROWS)])
    plsc.subcore_barrier()
    chunk_base = jnp.where(c == 0, s * CPT0, NS * CPT0 + s * CPT1)
    nblk = jnp.where(c == 0, CPT0 // IBLK, CPT1 // IBLK)

    def blk_body(blk, carry):
        base = pl.multiple_of(chunk_base + blk * IBLK, 8)
        pltpu.sync_copy(src_hbm.at[pl.ds(base, IBLK)], sidx)
        pltpu.sync_copy(dst_hbm.at[pl.ds(base, IBLK)], didx)
        for b in range(NBUF):
            pltpu.async_copy(g_hbm.at[sidx.at[b]], rows[b], sems[b])

        def body(i, _):
            for b in range(NBUF):
                j = i * NBUF + b
                pltpu.make_async_copy(
                    g_hbm.at[pl.ds(0, CHUNK)], rows[b], sems[b]).wait()
                pltpu.sync_copy(rows[b], acc.at[didx.at[j]], add=True)
                pltpu.async_copy(g_hbm.at[sidx.at[j + NBUF]], rows[b], sems[b])
            return _

        lax.fori_loop(0, IBLK // NBUF - 1, body, None)
        for b in range(NBUF):
            j = IBLK - NBUF + b
            pltpu.make_async_copy(
                g_hbm.at[pl.ds(0, CHUNK)], rows[b], sems[b]).wait()
            pltpu.sync_copy(rows[b], acc.at[didx.at[j]], add=True)
        return carry

    lax.fori_loop(0, nblk, blk_body, None)
    plsc.subcore_barrier()
    pltpu.sync_copy(acc.at[pl.ds(s * ROWS_OUT, ROWS_OUT)], out_hbm.at[c, s])


@functools.partial(
    pl.kernel,
    out_type=jax.ShapeDtypeStruct((NW, CPT, CHUNK), jnp.float32),
    mesh=_mesh,
    compiler_params=pltpu.CompilerParams(needs_layout_passes=False),
    scratch_types=[
        pltpu.VMEM((N_PAD,), jnp.float32),
        pltpu.VMEM((N_PAD,), jnp.float32),
        pltpu.VMEM((CPT, CHUNK), jnp.int32),
        pltpu.VMEM((CPT, CHUNK), jnp.int32),
        pltpu.VMEM((CPT, CHUNK), jnp.float32),
    ],
)
def _sc_edge_logits(a_hbm, b_hbm, src_hbm, dst_hbm, out_hbm,
                    a_v, b_v, sidx, didx, out_v):
    c = lax.axis_index("c")
    s = lax.axis_index("s")
    w = c * NS + s
    nreal = jnp.minimum(CPT, jnp.maximum(0, TOTAL_CHUNKS - w * CPT))
    pltpu.sync_copy(a_hbm, a_v)
    pltpu.sync_copy(b_hbm, b_v)
    pltpu.sync_copy(src_hbm.at[pl.ds(w * CPT, CPT)], sidx)
    pltpu.sync_copy(dst_hbm.at[pl.ds(w * CPT, CPT)], didx)

    def body(j, _):
        for k in range(CHUNK // 16):
            isrc = sidx[j, pl.ds(k * 16, 16)]
            idst = didx[j, pl.ds(k * 16, 16)]
            ag = plsc.load_gather(a_v, [isrc])
            bg = plsc.load_gather(b_v, [idst])
            t = ag + bg
            out_v[j, pl.ds(k * 16, 16)] = 1.0 / (1.0 + jnp.exp(-t))
        return _

    lax.fori_loop(0, nreal, body, None)
    pltpu.sync_copy(out_v, out_hbm.at[w])


_BLK = 1000
_GRID = N_NODES // _BLK


def _tc_a_body(x_ref, w1_ref, dp_ref, g_ref, dinv_ref):
    dp = dp_ref[...]
    deg = dp[0, :, :16] + dp[1, :, :16] + 1.0
    dinv = lax.rsqrt(deg)
    h = jnp.dot(x_ref[...], w1_ref[...], preferred_element_type=jnp.float32,
                 precision=lax.Precision.HIGHEST)
    g_ref[...] = h * dinv[:, :1]
    dinv_ref[...] = dinv


def _tc_b_body(part_ref, g_ref, dinv_ref, b_ref, w_ref, gout_ref):
    ssum = part_ref[0] + part_ref[1] + g_ref[...]
    dinv = dinv_ref[:, :1]
    z = jnp.maximum(ssum * dinv + b_ref[...], 0.0)
    gout_ref[...] = jnp.dot(z, w_ref[...], preferred_element_type=jnp.float32,
                 precision=lax.Precision.HIGHEST) * dinv


def _tc_c_body(part_ref, g_ref, dinv_ref, b_ref, wl_ref, bl_ref, a_ref, bv_ref):
    ssum = part_ref[0] + part_ref[1] + g_ref[...]
    dinv = dinv_ref[:, :1]
    z = jnp.maximum(ssum * dinv + b_ref[...], 0.0)
    a_ref[...] = jnp.sum(z * wl_ref[0:1, :], axis=1, keepdims=True) + bl_ref[...]
    bv_ref[...] = jnp.sum(z * wl_ref[1:2, :], axis=1, keepdims=True)


def _rows_spec():
    return pl.BlockSpec((_BLK, D), lambda i: (i, 0))


def _full_spec(shape):
    nd = len(shape)
    return pl.BlockSpec(shape, lambda i, _nd=nd: (0,) * _nd)


def _tc_a(x, w1, deg_parts):
    return pl.pallas_call(
        _tc_a_body,
        grid=(_GRID,),
        in_specs=[
            _rows_spec(),
            _full_spec((D, D)),
            pl.BlockSpec((2, _BLK, D), lambda i: (0, i, 0)),
        ],
        out_specs=[_rows_spec(), pl.BlockSpec((_BLK, 16), lambda i: (i, 0))],
        out_shape=[
            jax.ShapeDtypeStruct((N_NODES, D), jnp.float32),
            jax.ShapeDtypeStruct((N_NODES, 16), jnp.float32),
        ],
    )(x, w1, deg_parts)


def _tc_b(parts, g, dinv16, b, w):
    return pl.pallas_call(
        _tc_b_body,
        grid=(_GRID,),
        in_specs=[
            pl.BlockSpec((2, _BLK, D), lambda i: (0, i, 0)),
            _rows_spec(),
            pl.BlockSpec((_BLK, 16), lambda i: (i, 0)),
            _full_spec((1, D)),
            _full_spec((D, D)),
        ],
        out_specs=_rows_spec(),
        out_shape=jax.ShapeDtypeStruct((N_NODES, D), jnp.float32),
    )(parts, g, dinv16, b, w)


def _tc_c(parts, g, dinv16, b, wl2, bl):
    return pl.pallas_call(
        _tc_c_body,
        grid=(_GRID,),
        in_specs=[
            pl.BlockSpec((2, _BLK, D), lambda i: (0, i, 0)),
            _rows_spec(),
            pl.BlockSpec((_BLK, 16), lambda i: (i, 0)),
            _full_spec((1, D)),
            _full_spec((2, D)),
            _full_spec((1, 1)),
        ],
        out_specs=[
            pl.BlockSpec((_BLK, 1), lambda i: (i, 0)),
            pl.BlockSpec((_BLK, 1), lambda i: (i, 0)),
        ],
        out_shape=[
            jax.ShapeDtypeStruct((N_NODES, 1), jnp.float32),
            jax.ShapeDtypeStruct((N_NODES, 1), jnp.float32),
        ],
    )(parts, g, dinv16, b, wl2, bl)


def kernel(x, edge_index, W1, b1, W2, b2, Wl, bl):
    src = edge_index[0].astype(jnp.int32)
    dst = edge_index[1].astype(jnp.int32)
    npad = E_PAD - N_EDGES
    src_r = jnp.concatenate([src, jnp.zeros((npad,), jnp.int32)]).reshape(
        E_PAD // CHUNK, CHUNK)
    dst_r = jnp.concatenate([dst, jnp.full((npad,), N_NODES, jnp.int32)]).reshape(
        E_PAD // CHUNK, CHUNK)

    onesD = jnp.ones((CHUNK, D), jnp.float32)
    zerosD = jnp.zeros((ZROWS, D), jnp.float32)

    deg_parts = _sc_degree(dst_r, onesD, zerosD)
    deg_parts = deg_parts.reshape(NC, N_NODES, D)

    g1, dinv16 = _tc_a(x, W1, deg_parts)

    parts1 = _sc_aggregate(g1, src_r, dst_r, zerosD).reshape(NC, N_NODES, D)
    g2 = _tc_b(parts1, g1, dinv16, b1.reshape(1, D), W2)

    parts2 = _sc_aggregate(g2, src_r, dst_r, zerosD).reshape(NC, N_NODES, D)
    wl2 = Wl[:, 0].reshape(2, D)
    a_col, b_col = _tc_c(parts2, g2, dinv16, b2.reshape(1, D), wl2,
                         bl.reshape(1, 1))

    a_pad = jnp.pad(a_col.reshape(N_NODES), (0, N_PAD - N_NODES))
    b_pad = jnp.pad(b_col.reshape(N_NODES), (0, N_PAD - N_NODES))

    probs = _sc_edge_logits(a_pad, b_pad, src_r, dst_r)
    return probs.reshape(E_PAD)[:N_EDGES]


# speedup vs baseline: 1.1280x; 1.1280x over previous
import functools

import jax
import jax.numpy as jnp
from jax import lax
from jax.experimental import pallas as pl
from jax.experimental.pallas import tpu as pltpu
from jax.experimental.pallas import tpu_sc as plsc

N_NODES = 10000
D = 128
N_EDGES = 320000

NC = 2
NS = 16
NW = NC * NS

CHUNK = 128
CPT = 80
EPT = CHUNK * CPT
E_PAD = EPT * NW
ACC_ROWS = 10240
ZROWS = ACC_ROWS // NS
ROWS_OUT = N_NODES // NS
N_PAD = 10016
TOTAL_CHUNKS = N_EDGES // CHUNK
CPT0 = 80
CPT1 = 80
NBUF = 2
IBLK = 16
NIB = CPT // IBLK

_mesh = plsc.VectorSubcoreMesh(
    core_axis_name="c", subcore_axis_name="s", num_cores=NC, num_subcores=NS)


@functools.partial(
    pl.kernel,
    out_type=jax.ShapeDtypeStruct((NC, NS, ROWS_OUT, D), jnp.float32),
    mesh=_mesh,
    scratch_types=[
        pltpu.VMEM((CPT, CHUNK), jnp.int32),
        pltpu.VMEM((CHUNK, D), jnp.float32),
        pltpu.VMEM_SHARED((ACC_ROWS, D), jnp.float32),
    ],
)
def _sc_degree(dst_hbm, ones_hbm, zeros_hbm, out_hbm, idx_v, ones_v, acc):
    c = lax.axis_index("c")
    s = lax.axis_index("s")
    w = c * NS + s
    nreal = jnp.minimum(CPT, jnp.maximum(0, TOTAL_CHUNKS - w * CPT))
    pltpu.sync_copy(dst_hbm.at[pl.ds(w * CPT, CPT)], idx_v)
    pltpu.sync_copy(ones_hbm, ones_v)
    pltpu.sync_copy(zeros_hbm, acc.at[pl.ds(s * ZROWS, ZROWS)])
    plsc.subcore_barrier()

    def body(j, _):
        pltpu.sync_copy(ones_v, acc.at[idx_v.at[j]], add=True)
        return _

    lax.fori_loop(0, nreal, body, None)
    plsc.subcore_barrier()
    pltpu.sync_copy(acc.at[pl.ds(s * ROWS_OUT, ROWS_OUT)], out_hbm.at[c, s])


@functools.partial(
    pl.kernel,
    out_type=jax.ShapeDtypeStruct((NC, NS, ROWS_OUT, D), jnp.float32),
    mesh=_mesh,
    scratch_types=[
        pltpu.VMEM((IBLK, CHUNK), jnp.int32),
        pltpu.VMEM((IBLK, CHUNK), jnp.int32),
        pltpu.VMEM((CHUNK, D), jnp.float32),
        pltpu.VMEM((CHUNK, D), jnp.float32),
        pltpu.SemaphoreType.DMA,
        pltpu.SemaphoreType.DMA,
        pltpu.VMEM_SHARED((ACC_ROWS, D), jnp.float32),
    ],
)
def _sc_aggregate(g_hbm, src_hbm, dst_hbm, zeros_hbm, out_hbm,
                  sidx, didx, r0, r1, sm0, sm1, acc):
    rows = [r0, r1]
    sems = [sm0, sm1]
    c = lax.axis_index("c")
    s = lax.axis_index("s")
    pltpu.sync_copy(zeros_hbm, acc.at[pl.ds(s * ZROWS, ZROWS)])
    plsc.subcore_barrier()
    chunk_base = jnp.where(c == 0, s * CPT0, NS * CPT0 + s * CPT1)
    nblk = jnp.where(c == 0, CPT0 // IBLK, CPT1 // IBLK)

    def blk_body(blk, carry):
        base = pl.multiple_of(chunk_base + blk * IBLK, 8)
        pltpu.sync_copy(src_hbm.at[pl.ds(base, IBLK)], sidx)
        pltpu.sync_copy(dst_hbm.at[pl.ds(base, IBLK)], didx)
        for b in range(NBUF):
            pltpu.async_copy(g_hbm.at[sidx.at[b]], rows[b], sems[b])

        def body(i, _):
            for b in range(NBUF):
                j = i * NBUF + b
                pltpu.make_async_copy(
                    g_hbm.at[pl.ds(0, CHUNK)], rows[b], sems[b]).wait()
                pltpu.sync_copy(rows[b], acc.at[didx.at[j]], add=True)
                pltpu.async_copy(g_hbm.at[sidx.at[j + NBUF]], rows[b], sems[b])
            return _

        lax.fori_loop(0, IBLK // NBUF - 1, body, None)
        for b in range(NBUF):
            j = IBLK - NBUF + b
            pltpu.make_async_copy(
                g_hbm.at[pl.ds(0, CHUNK)], rows[b], sems[b]).wait()
            pltpu.sync_copy(rows[b], acc.at[didx.at[j]], add=True)
        return carry

    lax.fori_loop(0, nblk, blk_body, None)
    plsc.subcore_barrier()
    pltpu.sync_copy(acc.at[pl.ds(s * ROWS_OUT, ROWS_OUT)], out_hbm.at[c, s])


@functools.partial(
    pl.kernel,
    out_type=jax.ShapeDtypeStruct((NW, CPT, CHUNK), jnp.float32),
    mesh=_mesh,
    compiler_params=pltpu.CompilerParams(needs_layout_passes=False),
    scratch_types=[
        pltpu.VMEM((N_PAD,), jnp.float32),
        pltpu.VMEM((N_PAD,), jnp.float32),
        pltpu.VMEM((CPT, CHUNK), jnp.int32),
        pltpu.VMEM((CPT, CHUNK), jnp.int32),
        pltpu.VMEM((CPT, CHUNK), jnp.float32),
    ],
)
def _sc_edge_logits(a_hbm, b_hbm, src_hbm, dst_hbm, out_hbm,
                    a_v, b_v, sidx, didx, out_v):
    c = lax.axis_index("c")
    s = lax.axis_index("s")
    w = c * NS + s
    nreal = jnp.minimum(CPT, jnp.maximum(0, TOTAL_CHUNKS - w * CPT))
    pltpu.sync_copy(a_hbm, a_v)
    pltpu.sync_copy(b_hbm, b_v)
    pltpu.sync_copy(src_hbm.at[pl.ds(w * CPT, CPT)], sidx)
    pltpu.sync_copy(dst_hbm.at[pl.ds(w * CPT, CPT)], didx)

    def body(j, _):
        for k in range(CHUNK // 16):
            isrc = sidx[j, pl.ds(k * 16, 16)]
            idst = didx[j, pl.ds(k * 16, 16)]
            ag = plsc.load_gather(a_v, [isrc])
            bg = plsc.load_gather(b_v, [idst])
            t = ag + bg
            out_v[j, pl.ds(k * 16, 16)] = 1.0 / (1.0 + jnp.exp(-t))
        return _

    lax.fori_loop(0, nreal, body, None)
    pltpu.sync_copy(out_v, out_hbm.at[w])


_BLK = 1000
_GRID = N_NODES // _BLK


def _tc_a_body(x_ref, w1_ref, dp_ref, g_ref, dinv_ref):
    dp = dp_ref[...]
    deg = dp[0, :, :16] + dp[1, :, :16] + 1.0
    dinv = lax.rsqrt(deg)
    h = jnp.dot(x_ref[...], w1_ref[...], preferred_element_type=jnp.float32,
                 precision=lax.Precision.HIGHEST)
    g_ref[...] = h * dinv[:, :1]
    dinv_ref[...] = dinv


def _tc_b_body(part_ref, g_ref, dinv_ref, b_ref, w_ref, gout_ref):
    ssum = part_ref[0] + part_ref[1] + g_ref[...]
    dinv = dinv_ref[:, :1]
    z = jnp.maximum(ssum * dinv + b_ref[...], 0.0)
    gout_ref[...] = jnp.dot(z, w_ref[...], preferred_element_type=jnp.float32,
                 precision=lax.Precision.HIGHEST) * dinv


def _tc_c_body(part_ref, g_ref, dinv_ref, b_ref, wl_ref, bl_ref, a_ref, bv_ref):
    ssum = part_ref[0] + part_ref[1] + g_ref[...]
    dinv = dinv_ref[:, :1]
    z = jnp.maximum(ssum * dinv + b_ref[...], 0.0)
    a_ref[...] = jnp.sum(z * wl_ref[0:1, :], axis=1, keepdims=True) + bl_ref[...]
    bv_ref[...] = jnp.sum(z * wl_ref[1:2, :], axis=1, keepdims=True)


def _rows_spec():
    return pl.BlockSpec((_BLK, D), lambda i: (i, 0))


def _full_spec(shape):
    nd = len(shape)
    return pl.BlockSpec(shape, lambda i, _nd=nd: (0,) * _nd)


def _tc_a(x, w1, deg_parts):
    return pl.pallas_call(
        _tc_a_body,
        grid=(_GRID,),
        in_specs=[
            _rows_spec(),
            _full_spec((D, D)),
            pl.BlockSpec((2, _BLK, D), lambda i: (0, i, 0)),
        ],
        out_specs=[_rows_spec(), pl.BlockSpec((_BLK, 16), lambda i: (i, 0))],
        out_shape=[
            jax.ShapeDtypeStruct((N_NODES, D), jnp.float32),
            jax.ShapeDtypeStruct((N_NODES, 16), jnp.float32),
        ],
    )(x, w1, deg_parts)


def _tc_b(parts, g, dinv16, b, w):
    return pl.pallas_call(
        _tc_b_body,
        grid=(_GRID,),
        in_specs=[
            pl.BlockSpec((2, _BLK, D), lambda i: (0, i, 0)),
            _rows_spec(),
            pl.BlockSpec((_BLK, 16), lambda i: (i, 0)),
            _full_spec((1, D)),
            _full_spec((D, D)),
        ],
        out_specs=_rows_spec(),
        out_shape=jax.ShapeDtypeStruct((N_NODES, D), jnp.float32),
    )(parts, g, dinv16, b, w)


def _tc_c(parts, g, dinv16, b, wl2, bl):
    return pl.pallas_call(
        _tc_c_body,
        grid=(_GRID,),
        in_specs=[
            pl.BlockSpec((2, _BLK, D), lambda i: (0, i, 0)),
            _rows_spec(),
            pl.BlockSpec((_BLK, 16), lambda i: (i, 0)),
            _full_spec((1, D)),
            _full_spec((2, D)),
            _full_spec((1, 1)),
        ],
        out_specs=[
            pl.BlockSpec((_BLK, 1), lambda i: (i, 0)),
            pl.BlockSpec((_BLK, 1), lambda i: (i, 0)),
        ],
        out_shape=[
            jax.ShapeDtypeStruct((N_NODES, 1), jnp.float32),
            jax.ShapeDtypeStruct((N_NODES, 1), jnp.float32),
        ],
    )(parts, g, dinv16, b, wl2, bl)


def kernel(x, edge_index, W1, b1, W2, b2, Wl, bl):
    src = edge_index[0].astype(jnp.int32)
    dst = edge_index[1].astype(jnp.int32)
    npad = E_PAD - N_EDGES
    src_r = jnp.concatenate([src, jnp.zeros((npad,), jnp.int32)]).reshape(
        E_PAD // CHUNK, CHUNK)
    fake_dst = N_NODES + (jnp.arange(npad, dtype=jnp.int32) % (ACC_ROWS - N_NODES))
    dst_r = jnp.concatenate([dst, fake_dst]).reshape(E_PAD // CHUNK, CHUNK)

    onesD = jnp.ones((CHUNK, D), jnp.float32)
    zerosD = jnp.zeros((ZROWS, D), jnp.float32)

    deg_parts = _sc_degree(dst_r, onesD, zerosD)
    deg_parts = deg_parts.reshape(NC, N_NODES, D)

    g1, dinv16 = _tc_a(x, W1, deg_parts)

    parts1 = _sc_aggregate(g1, src_r, dst_r, zerosD).reshape(NC, N_NODES, D)
    g2 = _tc_b(parts1, g1, dinv16, b1.reshape(1, D), W2)

    parts2 = _sc_aggregate(g2, src_r, dst_r, zerosD).reshape(NC, N_NODES, D)
    wl2 = Wl[:, 0].reshape(2, D)
    a_col, b_col = _tc_c(parts2, g2, dinv16, b2.reshape(1, D), wl2,
                         bl.reshape(1, 1))

    a_pad = jnp.pad(a_col.reshape(N_NODES), (0, N_PAD - N_NODES))
    b_pad = jnp.pad(b_col.reshape(N_NODES), (0, N_PAD - N_NODES))

    probs = _sc_edge_logits(a_pad, b_pad, src_r, dst_r)
    return probs.reshape(E_PAD)[:N_EDGES]


# speedup vs baseline: 3.1025x; 2.7505x over previous
import functools

import jax
import jax.numpy as jnp
from jax import lax
from jax.experimental import pallas as pl
from jax.experimental.pallas import tpu as pltpu
from jax.experimental.pallas import tpu_sc as plsc

N_NODES = 10000
D = 128
N_EDGES = 320000

NC = 2
NS = 16
NW = NC * NS

CHUNK = 128
CPT = 80
EPT = CHUNK * CPT
E_PAD = EPT * NW
ACC_ROWS = 10240
ZROWS = ACC_ROWS // NS
ROWS_OUT = N_NODES // NS
N_PAD = 10016
TOTAL_CHUNKS = N_EDGES // CHUNK
CPT0 = 80
CPT1 = 80
NBUF = 2
IBLK = 16
NIB = CPT // IBLK

_mesh = plsc.VectorSubcoreMesh(
    core_axis_name="c", subcore_axis_name="s", num_cores=NC, num_subcores=NS)


@functools.partial(
    pl.kernel,
    out_type=jax.ShapeDtypeStruct((NC, NS, ROWS_OUT, D), jnp.float32),
    mesh=_mesh,
    scratch_types=[
        pltpu.VMEM((CPT, CHUNK), jnp.int32),
        pltpu.VMEM((CHUNK, D), jnp.float32),
        pltpu.VMEM_SHARED((ACC_ROWS, D), jnp.float32),
    ],
)
def _sc_degree(dst_hbm, ones_hbm, zeros_hbm, out_hbm, idx_v, ones_v, acc):
    c = lax.axis_index("c")
    s = lax.axis_index("s")
    w = c * NS + s
    nreal = jnp.minimum(CPT, jnp.maximum(0, TOTAL_CHUNKS - w * CPT))
    pltpu.sync_copy(dst_hbm.at[pl.ds(w * CPT, CPT)], idx_v)
    pltpu.sync_copy(ones_hbm, ones_v)
    pltpu.sync_copy(zeros_hbm, acc.at[pl.ds(s * ZROWS, ZROWS)])
    plsc.subcore_barrier()

    def body(j, _):
        pltpu.sync_copy(ones_v, acc.at[idx_v.at[j]], add=True)
        return _

    lax.fori_loop(0, nreal, body, None)
    plsc.subcore_barrier()
    pltpu.sync_copy(acc.at[pl.ds(s * ROWS_OUT, ROWS_OUT)], out_hbm.at[c, s])


@functools.partial(
    pl.kernel,
    out_type=jax.ShapeDtypeStruct((NC, NS, ROWS_OUT, D), jnp.float32),
    mesh=_mesh,
    scratch_types=[
        pltpu.VMEM((IBLK, CHUNK), jnp.int32),
        pltpu.VMEM((IBLK, CHUNK), jnp.int32),
        pltpu.VMEM((CHUNK, D), jnp.float32),
        pltpu.VMEM((CHUNK, D), jnp.float32),
        pltpu.SemaphoreType.DMA,
        pltpu.SemaphoreType.DMA,
        pltpu.VMEM_SHARED((ACC_ROWS, D), jnp.float32),
    ],
)
def _sc_aggregate(g_hbm, src_hbm, dst_hbm, zeros_hbm, out_hbm,
                  sidx, didx, r0, r1, sm0, sm1, acc):
    rows = [r0, r1]
    sems = [sm0, sm1]
    c = lax.axis_index("c")
    s = lax.axis_index("s")
    pltpu.sync_copy(zeros_hbm, acc.at[pl.ds(s * ZROWS, ZROWS)])
    plsc.subcore_barrier()
    chunk_base = jnp.where(c == 0, s * CPT0, NS * CPT0 + s * CPT1)
    nblk = jnp.where(c == 0, CPT0 // IBLK, CPT1 // IBLK)

    def blk_body(blk, carry):
        base = pl.multiple_of(chunk_base + blk * IBLK, 8)
        pltpu.sync_copy(src_hbm.at[pl.ds(base, IBLK)], sidx)
        pltpu.sync_copy(dst_hbm.at[pl.ds(base, IBLK)], didx)
        for b in range(NBUF):
            pltpu.async_copy(g_hbm.at[sidx.at[b]], rows[b], sems[b])

        def body(i, _):
            for b in range(NBUF):
                j = i * NBUF + b
                pltpu.make_async_copy(
                    g_hbm.at[pl.ds(0, CHUNK)], rows[b], sems[b]).wait()
                pltpu.sync_copy(rows[b], acc.at[didx.at[j]], add=True)
                pltpu.async_copy(g_hbm.at[sidx.at[j + NBUF]], rows[b], sems[b])
            return _

        lax.fori_loop(0, IBLK // NBUF - 1, body, None)
        for b in range(NBUF):
            j = IBLK - NBUF + b
            pltpu.make_async_copy(
                g_hbm.at[pl.ds(0, CHUNK)], rows[b], sems[b]).wait()
            pltpu.sync_copy(rows[b], acc.at[didx.at[j]], add=True)
        return carry

    lax.fori_loop(0, nblk, blk_body, None)
    plsc.subcore_barrier()
    pltpu.sync_copy(acc.at[pl.ds(s * ROWS_OUT, ROWS_OUT)], out_hbm.at[c, s])


@functools.partial(
    pl.kernel,
    out_type=jax.ShapeDtypeStruct((NW, CPT, CHUNK), jnp.float32),
    mesh=_mesh,
    compiler_params=pltpu.CompilerParams(needs_layout_passes=False),
    scratch_types=[
        pltpu.VMEM((N_PAD,), jnp.float32),
        pltpu.VMEM((N_PAD,), jnp.float32),
        pltpu.VMEM((CPT, CHUNK), jnp.int32),
        pltpu.VMEM((CPT, CHUNK), jnp.int32),
        pltpu.VMEM((CPT, CHUNK), jnp.float32),
    ],
)
def _sc_edge_logits(a_hbm, b_hbm, src_hbm, dst_hbm, out_hbm,
                    a_v, b_v, sidx, didx, out_v):
    c = lax.axis_index("c")
    s = lax.axis_index("s")
    w = c * NS + s
    nreal = jnp.minimum(CPT, jnp.maximum(0, TOTAL_CHUNKS - w * CPT))
    pltpu.sync_copy(a_hbm, a_v)
    pltpu.sync_copy(b_hbm, b_v)
    pltpu.sync_copy(src_hbm.at[pl.ds(w * CPT, CPT)], sidx)
    pltpu.sync_copy(dst_hbm.at[pl.ds(w * CPT, CPT)], didx)

    def body(j, _):
        for k in range(CHUNK // 16):
            isrc = sidx[j, pl.ds(k * 16, 16)]
            idst = didx[j, pl.ds(k * 16, 16)]
            ag = plsc.load_gather(a_v, [isrc])
            bg = plsc.load_gather(b_v, [idst])
            t = ag + bg
            out_v[j, pl.ds(k * 16, 16)] = 1.0 / (1.0 + jnp.exp(-t))
        return _

    lax.fori_loop(0, nreal, body, None)
    pltpu.sync_copy(out_v, out_hbm.at[w])


_BLK = 1000
_GRID = N_NODES // _BLK


def _tc_a_body(x_ref, w1_ref, dp_ref, g_ref, dinv_ref):
    dp = dp_ref[...]
    deg = dp[0, :, :16] + dp[1, :, :16] + 1.0
    dinv = lax.rsqrt(deg)
    h = jnp.dot(x_ref[...], w1_ref[...], preferred_element_type=jnp.float32,
                 precision=lax.Precision.HIGHEST)
    g_ref[...] = h * dinv[:, :1]
    dinv_ref[...] = dinv


def _tc_b_body(part_ref, g_ref, dinv_ref, b_ref, w_ref, gout_ref):
    ssum = part_ref[0] + part_ref[1] + g_ref[...]
    dinv = dinv_ref[:, :1]
    z = jnp.maximum(ssum * dinv + b_ref[...], 0.0)
    gout_ref[...] = jnp.dot(z, w_ref[...], preferred_element_type=jnp.float32,
                 precision=lax.Precision.HIGHEST) * dinv


def _tc_c_body(part_ref, g_ref, dinv_ref, b_ref, wl_ref, bl_ref, a_ref, bv_ref):
    ssum = part_ref[0] + part_ref[1] + g_ref[...]
    dinv = dinv_ref[:, :1]
    z = jnp.maximum(ssum * dinv + b_ref[...], 0.0)
    a_ref[...] = jnp.sum(z * wl_ref[0:1, :], axis=1, keepdims=True) + bl_ref[...]
    bv_ref[...] = jnp.sum(z * wl_ref[1:2, :], axis=1, keepdims=True)


def _rows_spec():
    return pl.BlockSpec((_BLK, D), lambda i: (i, 0))


def _full_spec(shape):
    nd = len(shape)
    return pl.BlockSpec(shape, lambda i, _nd=nd: (0,) * _nd)


def _tc_a(x, w1, deg_parts):
    return pl.pallas_call(
        _tc_a_body,
        grid=(_GRID,),
        in_specs=[
            _rows_spec(),
            _full_spec((D, D)),
            pl.BlockSpec((2, _BLK, D), lambda i: (0, i, 0)),
        ],
        out_specs=[_rows_spec(), pl.BlockSpec((_BLK, 16), lambda i: (i, 0))],
        out_shape=[
            jax.ShapeDtypeStruct((N_NODES, D), jnp.float32),
            jax.ShapeDtypeStruct((N_NODES, 16), jnp.float32),
        ],
    )(x, w1, deg_parts)


def _tc_b(parts, g, dinv16, b, w):
    return pl.pallas_call(
        _tc_b_body,
        grid=(_GRID,),
        in_specs=[
            pl.BlockSpec((2, _BLK, D), lambda i: (0, i, 0)),
            _rows_spec(),
            pl.BlockSpec((_BLK, 16), lambda i: (i, 0)),
            _full_spec((1, D)),
            _full_spec((D, D)),
        ],
        out_specs=_rows_spec(),
        out_shape=jax.ShapeDtypeStruct((N_NODES, D), jnp.float32),
    )(parts, g, dinv16, b, w)


def _tc_c(parts, g, dinv16, b, wl2, bl):
    return pl.pallas_call(
        _tc_c_body,
        grid=(_GRID,),
        in_specs=[
            pl.BlockSpec((2, _BLK, D), lambda i: (0, i, 0)),
            _rows_spec(),
            pl.BlockSpec((_BLK, 16), lambda i: (i, 0)),
            _full_spec((1, D)),
            _full_spec((2, D)),
            _full_spec((1, 1)),
        ],
        out_specs=[
            pl.BlockSpec((_BLK, 1), lambda i: (i, 0)),
            pl.BlockSpec((_BLK, 1), lambda i: (i, 0)),
        ],
        out_shape=[
            jax.ShapeDtypeStruct((N_NODES, 1), jnp.float32),
            jax.ShapeDtypeStruct((N_NODES, 1), jnp.float32),
        ],
    )(parts, g, dinv16, b, wl2, bl)


def kernel(x, edge_index, W1, b1, W2, b2, Wl, bl):
    src = edge_index[0].astype(jnp.int32)
    dst = edge_index[1].astype(jnp.int32)
    npad = E_PAD - N_EDGES
    src_r = jnp.concatenate([src, src[:npad]]).reshape(E_PAD // CHUNK, CHUNK)
    fake_dst = N_NODES + (jnp.arange(npad, dtype=jnp.int32) % (ACC_ROWS - N_NODES))
    dst_r = jnp.concatenate([dst, fake_dst]).reshape(E_PAD // CHUNK, CHUNK)

    onesD = jnp.ones((CHUNK, D), jnp.float32)
    zerosD = jnp.zeros((ZROWS, D), jnp.float32)

    deg_parts = _sc_degree(dst_r, onesD, zerosD)
    deg_parts = deg_parts.reshape(NC, N_NODES, D)

    g1, dinv16 = _tc_a(x, W1, deg_parts)

    parts1 = _sc_aggregate(g1, src_r, dst_r, zerosD).reshape(NC, N_NODES, D)
    g2 = _tc_b(parts1, g1, dinv16, b1.reshape(1, D), W2)

    parts2 = _sc_aggregate(g2, src_r, dst_r, zerosD).reshape(NC, N_NODES, D)
    wl2 = Wl[:, 0].reshape(2, D)
    a_col, b_col = _tc_c(parts2, g2, dinv16, b2.reshape(1, D), wl2,
                         bl.reshape(1, 1))

    a_pad = jnp.pad(a_col.reshape(N_NODES), (0, N_PAD - N_NODES))
    b_pad = jnp.pad(b_col.reshape(N_NODES), (0, N_PAD - N_NODES))

    probs = _sc_edge_logits(a_pad, b_pad, src_r, dst_r)
    return probs.reshape(E_PAD)[:N_EDGES]


# speedup vs baseline: 3.6223x; 1.1676x over previous
import functools

import jax
import jax.numpy as jnp
from jax import lax
from jax.experimental import pallas as pl
from jax.experimental.pallas import tpu as pltpu
from jax.experimental.pallas import tpu_sc as plsc

N_NODES = 10000
D = 128
N_EDGES = 320000

NC = 2
NS = 16
NW = NC * NS

CHUNK = 128
CPT = 80
EPT = CHUNK * CPT
E_PAD = EPT * NW
ACC_ROWS = 10240
ZROWS = ACC_ROWS // NS
ROWS_OUT = N_NODES // NS
N_PAD = 10016
TOTAL_CHUNKS = N_EDGES // CHUNK
CPT0 = 80
CPT1 = 80
NBUF = 2
IBLK = 16
NIB = CPT // IBLK

_mesh = plsc.VectorSubcoreMesh(
    core_axis_name="c", subcore_axis_name="s", num_cores=NC, num_subcores=NS)


HROWS = 80


@functools.partial(
    pl.kernel,
    out_type=jax.ShapeDtypeStruct((NC, HROWS, D), jnp.float32),
    mesh=_mesh,
    compiler_params=pltpu.CompilerParams(needs_layout_passes=False),
    scratch_types=[
        pltpu.VMEM((CPT, CHUNK), jnp.int32),
        pltpu.VMEM((HROWS, D), jnp.float32),
        pltpu.VMEM((HROWS,), jnp.int32),
        pltpu.VMEM_SHARED((HROWS, D), jnp.float32),
    ],
)
def _sc_degree(dst_hbm, zeros_hbm, out_hbm, idx_v, hist, rowidx, acc):
    c = lax.axis_index("c")
    s = lax.axis_index("s")
    w = c * NS + s
    nreal = jnp.minimum(CPT, jnp.maximum(0, TOTAL_CHUNKS - w * CPT))
    pltpu.sync_copy(dst_hbm.at[pl.ds(w * CPT, CPT)], idx_v)
    pltpu.sync_copy(zeros_hbm.at[pl.ds(0, HROWS)], hist)
    for m in range(HROWS // 16):
        rowidx[pl.ds(m * 16, 16)] = lax.iota(jnp.int32, 16) + m * 16
    @pl.when(s == 0)
    def _():
        pltpu.sync_copy(zeros_hbm.at[pl.ds(0, HROWS)], acc)
    plsc.subcore_barrier()

    ones16 = jnp.full((16,), 1.0, jnp.float32)

    def body(j, _):
        for k in range(CHUNK // 16):
            idx16 = idx_v[j, pl.ds(k * 16, 16)]
            plsc.addupdate_scatter(
                hist, [lax.shift_right_logical(idx16, 7),
                       lax.bitwise_and(idx16, 127)], ones16)
        return _

    lax.fori_loop(0, nreal, body, None)
    pltpu.sync_copy(hist, acc.at[rowidx], add=True)
    plsc.subcore_barrier()

    @pl.when(s == 0)
    def _():
        pltpu.sync_copy(acc, out_hbm.at[c])


@functools.partial(
    pl.kernel,
    out_type=jax.ShapeDtypeStruct((NC, NS, ROWS_OUT, D), jnp.float32),
    mesh=_mesh,
    scratch_types=[
        pltpu.VMEM((IBLK, CHUNK), jnp.int32),
        pltpu.VMEM((IBLK, CHUNK), jnp.int32),
        pltpu.VMEM((CHUNK, D), jnp.float32),
        pltpu.VMEM((CHUNK, D), jnp.float32),
        pltpu.SemaphoreType.DMA,
        pltpu.SemaphoreType.DMA,
        pltpu.VMEM_SHARED((ACC_ROWS, D), jnp.float32),
    ],
)
def _sc_aggregate(g_hbm, src_hbm, dst_hbm, zeros_hbm, out_hbm,
                  sidx, didx, r0, r1, sm0, sm1, acc):
    rows = [r0, r1]
    sems = [sm0, sm1]
    c = lax.axis_index("c")
    s = lax.axis_index("s")
    pltpu.sync_copy(zeros_hbm, acc.at[pl.ds(s * ZROWS, ZROWS)])
    plsc.subcore_barrier()
    chunk_base = jnp.where(c == 0, s * CPT0, NS * CPT0 + s * CPT1)
    nblk = jnp.where(c == 0, CPT0 // IBLK, CPT1 // IBLK)

    def blk_body(blk, carry):
        base = pl.multiple_of(chunk_base + blk * IBLK, 8)
        pltpu.sync_copy(src_hbm.at[pl.ds(base, IBLK)], sidx)
        pltpu.sync_copy(dst_hbm.at[pl.ds(base, IBLK)], didx)
        for b in range(NBUF):
            pltpu.async_copy(g_hbm.at[sidx.at[b]], rows[b], sems[b])

        def body(i, _):
            for b in range(NBUF):
                j = i * NBUF + b
                pltpu.make_async_copy(
                    g_hbm.at[pl.ds(0, CHUNK)], rows[b], sems[b]).wait()
                pltpu.sync_copy(rows[b], acc.at[didx.at[j]], add=True)
                pltpu.async_copy(g_hbm.at[sidx.at[j + NBUF]], rows[b], sems[b])
            return _

        lax.fori_loop(0, IBLK // NBUF - 1, body, None)
        for b in range(NBUF):
            j = IBLK - NBUF + b
            pltpu.make_async_copy(
                g_hbm.at[pl.ds(0, CHUNK)], rows[b], sems[b]).wait()
            pltpu.sync_copy(rows[b], acc.at[didx.at[j]], add=True)
        return carry

    lax.fori_loop(0, nblk, blk_body, None)
    plsc.subcore_barrier()
    pltpu.sync_copy(acc.at[pl.ds(s * ROWS_OUT, ROWS_OUT)], out_hbm.at[c, s])


@functools.partial(
    pl.kernel,
    out_type=jax.ShapeDtypeStruct((NW, CPT, CHUNK), jnp.float32),
    mesh=_mesh,
    compiler_params=pltpu.CompilerParams(needs_layout_passes=False),
    scratch_types=[
        pltpu.VMEM((N_PAD,), jnp.float32),
        pltpu.VMEM((N_PAD,), jnp.float32),
        pltpu.VMEM((CPT, CHUNK), jnp.int32),
        pltpu.VMEM((CPT, CHUNK), jnp.int32),
        pltpu.VMEM((CPT, CHUNK), jnp.float32),
    ],
)
def _sc_edge_logits(a_hbm, b_hbm, src_hbm, dst_hbm, out_hbm,
                    a_v, b_v, sidx, didx, out_v):
    c = lax.axis_index("c")
    s = lax.axis_index("s")
    w = c * NS + s
    nreal = jnp.minimum(CPT, jnp.maximum(0, TOTAL_CHUNKS - w * CPT))
    pltpu.sync_copy(a_hbm, a_v)
    pltpu.sync_copy(b_hbm, b_v)
    pltpu.sync_copy(src_hbm.at[pl.ds(w * CPT, CPT)], sidx)
    pltpu.sync_copy(dst_hbm.at[pl.ds(w * CPT, CPT)], didx)

    def body(j, _):
        for k in range(CHUNK // 16):
            isrc = sidx[j, pl.ds(k * 16, 16)]
            idst = didx[j, pl.ds(k * 16, 16)]
            ag = plsc.load_gather(a_v, [isrc])
            bg = plsc.load_gather(b_v, [idst])
            t = ag + bg
            out_v[j, pl.ds(k * 16, 16)] = 1.0 / (1.0 + jnp.exp(-t))
        return _

    lax.fori_loop(0, nreal, body, None)
    pltpu.sync_copy(out_v, out_hbm.at[w])


_BLK = 1000
_GRID = N_NODES // _BLK


def _tc_h_body(x_ref, w1_ref, h_ref):
    h_ref[...] = jnp.dot(x_ref[...], w1_ref[...],
                         preferred_element_type=jnp.float32,
                         precision=lax.Precision.HIGHEST)


def _tc_a_body(h_ref, dp_ref, g_ref, dinv_ref):
    deg = dp_ref[0] + dp_ref[1] + 1.0
    dinv = lax.rsqrt(deg)
    g_ref[...] = h_ref[...] * dinv[:, :1]
    dinv_ref[...] = jnp.broadcast_to(dinv, (dinv.shape[0], 16))


def _tc_b_body(part_ref, g_ref, dinv_ref, b_ref, w_ref, gout_ref):
    ssum = part_ref[0] + part_ref[1] + g_ref[...]
    dinv = dinv_ref[:, :1]
    z = jnp.maximum(ssum * dinv + b_ref[...], 0.0)
    gout_ref[...] = jnp.dot(z, w_ref[...], preferred_element_type=jnp.float32,
                 precision=lax.Precision.HIGHEST) * dinv


def _tc_c_body(part_ref, g_ref, dinv_ref, b_ref, wl_ref, bl_ref, a_ref, bv_ref):
    ssum = part_ref[0] + part_ref[1] + g_ref[...]
    dinv = dinv_ref[:, :1]
    z = jnp.maximum(ssum * dinv + b_ref[...], 0.0)
    a_ref[...] = jnp.sum(z * wl_ref[0:1, :], axis=1, keepdims=True) + bl_ref[...]
    bv_ref[...] = jnp.sum(z * wl_ref[1:2, :], axis=1, keepdims=True)


def _rows_spec():
    return pl.BlockSpec((_BLK, D), lambda i: (i, 0))


def _full_spec(shape):
    nd = len(shape)
    return pl.BlockSpec(shape, lambda i, _nd=nd: (0,) * _nd)


def _tc_h(x, w1):
    return pl.pallas_call(
        _tc_h_body,
        grid=(_GRID,),
        in_specs=[_rows_spec(), _full_spec((D, D))],
        out_specs=_rows_spec(),
        out_shape=jax.ShapeDtypeStruct((N_NODES, D), jnp.float32),
    )(x, w1)


def _tc_a(h, deg_col):
    return pl.pallas_call(
        _tc_a_body,
        grid=(_GRID,),
        in_specs=[
            _rows_spec(),
            pl.BlockSpec((2, _BLK, 1), lambda i: (0, i, 0)),
        ],
        out_specs=[_rows_spec(), pl.BlockSpec((_BLK, 16), lambda i: (i, 0))],
        out_shape=[
            jax.ShapeDtypeStruct((N_NODES, D), jnp.float32),
            jax.ShapeDtypeStruct((N_NODES, 16), jnp.float32),
        ],
    )(h, deg_col)


def _tc_b(parts, g, dinv16, b, w):
    return pl.pallas_call(
        _tc_b_body,
        grid=(_GRID,),
        in_specs=[
            pl.BlockSpec((2, _BLK, D), lambda i: (0, i, 0)),
            _rows_spec(),
            pl.BlockSpec((_BLK, 16), lambda i: (i, 0)),
            _full_spec((1, D)),
            _full_spec((D, D)),
        ],
        out_specs=_rows_spec(),
        out_shape=jax.ShapeDtypeStruct((N_NODES, D), jnp.float32),
    )(parts, g, dinv16, b, w)


def _tc_c(parts, g, dinv16, b, wl2, bl):
    return pl.pallas_call(
        _tc_c_body,
        grid=(_GRID,),
        in_specs=[
            pl.BlockSpec((2, _BLK, D), lambda i: (0, i, 0)),
            _rows_spec(),
            pl.BlockSpec((_BLK, 16), lambda i: (i, 0)),
            _full_spec((1, D)),
            _full_spec((2, D)),
            _full_spec((1, 1)),
        ],
        out_specs=[
            pl.BlockSpec((_BLK, 1), lambda i: (i, 0)),
            pl.BlockSpec((_BLK, 1), lambda i: (i, 0)),
        ],
        out_shape=[
            jax.ShapeDtypeStruct((N_NODES, 1), jnp.float32),
            jax.ShapeDtypeStruct((N_NODES, 1), jnp.float32),
        ],
    )(parts, g, dinv16, b, wl2, bl)


def kernel(x, edge_index, W1, b1, W2, b2, Wl, bl):
    src = edge_index[0].astype(jnp.int32)
    dst = edge_index[1].astype(jnp.int32)
    npad = E_PAD - N_EDGES
    src_r = jnp.concatenate([src, src[:npad]]).reshape(E_PAD // CHUNK, CHUNK)
    fake_dst = N_NODES + (jnp.arange(npad, dtype=jnp.int32) % (ACC_ROWS - N_NODES))
    dst_r = jnp.concatenate([dst, fake_dst]).reshape(E_PAD // CHUNK, CHUNK)

    zerosD = jnp.zeros((ZROWS, D), jnp.float32)

    deg_parts = _sc_degree(dst_r, zerosD)
    deg_col = deg_parts.reshape(NC, HROWS * D)[:, :N_NODES].reshape(
        NC, N_NODES, 1)

    h1 = _tc_h(x, W1)
    g1, dinv16 = _tc_a(h1, deg_col)

    parts1 = _sc_aggregate(g1, src_r, dst_r, zerosD).reshape(NC, N_NODES, D)
    g2 = _tc_b(parts1, g1, dinv16, b1.reshape(1, D), W2)

    parts2 = _sc_aggregate(g2, src_r, dst_r, zerosD).reshape(NC, N_NODES, D)
    wl2 = Wl[:, 0].reshape(2, D)
    a_col, b_col = _tc_c(parts2, g2, dinv16, b2.reshape(1, D), wl2,
                         bl.reshape(1, 1))

    a_pad = jnp.pad(a_col.reshape(N_NODES), (0, N_PAD - N_NODES))
    b_pad = jnp.pad(b_col.reshape(N_NODES), (0, N_PAD - N_NODES))

    probs = _sc_edge_logits(a_pad, b_pad, src_r, dst_r)
    return probs.reshape(E_PAD)[:N_EDGES]
